# Initial kernel scaffold; baseline (speedup 1.0000x reference)
import jax
import jax.numpy as jnp
from jax.experimental import pallas as pl


def kernel(mission_coords, edge_index, batch, uavs_info, action_mask, speeds, dist_matrix, timetogo_matrix, params):
    raise NotImplementedError("write your pallas kernel here")


# speedup vs baseline: 22.2610x; 22.2610x over previous
import functools

import jax
import jax.numpy as jnp
import numpy as np
from jax import lax
from jax.experimental import pallas as pl
from jax.experimental.pallas import tpu as pltpu
from jax.experimental.pallas import tpu_sc as plsc

NUM_UAVS = 50
NUM_MISSIONS = 1000
N_NODES = NUM_UAVS * NUM_MISSIONS
N_EDGES = 800000
HEADS = 4
HID = 16
EMB = 64
IN_CH = 6

NC = 2
NS = 16
NTILES = NC * NS
CHUNK = 128
CHUNKS_PER_TILE = 196
EDGES_PER_TILE = CHUNK * CHUNKS_PER_TILE
E_PAD = EDGES_PER_TILE * NTILES
NRANGE = 3128
NRANGE_LAST = N_NODES - 15 * NRANGE
BN = 1000
GRID_N = N_NODES // BN


def _write_qkvs(y, outs):
    for h in range(HEADS):
        outs[h][...] = y[:, 16 * h:16 * h + 16]
        outs[HEADS + h][...] = y[:, 64 + 16 * h:64 + 16 * h + 16]
        outs[2 * HEADS + h][...] = y[:, 128 + 16 * h:128 + 16 * h + 16]
    outs[3 * HEADS][...] = y[:, 192:256]


def _mm_first_body(x_ref, w_ref, b_ref, *outs):
    y = jnp.dot(x_ref[...], w_ref[...], preferred_element_type=jnp.float32, precision=lax.Precision.HIGHEST)
    y = y + b_ref[...]
    _write_qkvs(y, outs)


def _merge_acc(acc_ref, sprev_ref):
    cols = []
    for h in range(HEADS):
        a0 = acc_ref[0, h]
        a1 = acc_ref[1, h]
        num = a0[:, 0:16] + a1[:, 0:16]
        den = a0[:, 16:17] + a1[:, 16:17]
        cols.append(num / (den + 1e-16))
    x = jnp.concatenate(cols, axis=1) + sprev_ref[...]
    return jnp.maximum(x, 0.0)


def _mm_mid_body(acc_ref, sprev_ref, w_ref, b_ref, *outs):
    x = _merge_acc(acc_ref, sprev_ref)
    y = jnp.dot(x, w_ref[...], preferred_element_type=jnp.float32, precision=lax.Precision.HIGHEST) + b_ref[...]
    _write_qkvs(y, outs)


def _pool_body(acc_ref, sprev_ref, out_ref):
    x = _merge_acc(acc_ref, sprev_ref)
    p = jnp.sum(x, axis=0, keepdims=True) * (1.0 / NUM_MISSIONS)
    out_ref[...] = p.reshape(1, 1, 64)


def _head_body(pooled_ref, uinfo_ref, speeds_ref,
               wout_ref, bout_ref,
               wa1_ref, ba1_ref, wa2_ref, ba2_ref, wa3_ref, ba3_ref,
               wc1_ref, bc1_ref, wc2_ref, bc2_ref, wc3_ref, bc3_ref,
               logits_ref, val_ref):
    emb = jnp.dot(pooled_ref[...], wout_ref[...],
                  preferred_element_type=jnp.float32, precision=lax.Precision.HIGHEST) + bout_ref[...]
    comb = jnp.concatenate([uinfo_ref[...], emb, speeds_ref[...]], axis=1)

    h = jnp.maximum(jnp.dot(comb, wa1_ref[...], preferred_element_type=jnp.float32, precision=lax.Precision.HIGHEST) + ba1_ref[...], 0.0)
    h = jnp.maximum(jnp.dot(h, wa2_ref[...], preferred_element_type=jnp.float32, precision=lax.Precision.HIGHEST) + ba2_ref[...], 0.0)
    logits_ref[...] = jnp.dot(h, wa3_ref[...], preferred_element_type=jnp.float32, precision=lax.Precision.HIGHEST) + ba3_ref[...]

    h = jnp.maximum(jnp.dot(comb, wc1_ref[...], preferred_element_type=jnp.float32, precision=lax.Precision.HIGHEST) + bc1_ref[...], 0.0)
    h = jnp.maximum(jnp.dot(h, wc2_ref[...], preferred_element_type=jnp.float32, precision=lax.Precision.HIGHEST) + bc2_ref[...], 0.0)
    val_ref[...] = jnp.dot(h, wc3_ref[...], preferred_element_type=jnp.float32, precision=lax.Precision.HIGHEST) + bc3_ref[...]


def _qkvs_out_shapes():
    shapes = [jax.ShapeDtypeStruct((N_NODES, 16), jnp.float32) for _ in range(12)]
    shapes.append(jax.ShapeDtypeStruct((N_NODES, 64), jnp.float32))
    return shapes


def _qkvs_out_specs():
    specs = [pl.BlockSpec((BN, 16), lambda i: (i, 0)) for _ in range(12)]
    specs.append(pl.BlockSpec((BN, 64), lambda i: (i, 0)))
    return specs


def _mm_first(x, w, b):
    return pl.pallas_call(
        _mm_first_body,
        grid=(GRID_N,),
        in_specs=[
            pl.BlockSpec((BN, 8), lambda i: (i, 0)),
            pl.BlockSpec((8, 256), lambda i: (0, 0)),
            pl.BlockSpec((1, 256), lambda i: (0, 0)),
        ],
        out_specs=_qkvs_out_specs(),
        out_shape=_qkvs_out_shapes(),
    )(x, w, b)


def _mm_mid(acc, sprev, w, b):
    return pl.pallas_call(
        _mm_mid_body,
        grid=(GRID_N,),
        in_specs=[
            pl.BlockSpec((2, HEADS, BN, 32), lambda i: (0, 0, i, 0)),
            pl.BlockSpec((BN, 64), lambda i: (i, 0)),
            pl.BlockSpec((64, 256), lambda i: (0, 0)),
            pl.BlockSpec((1, 256), lambda i: (0, 0)),
        ],
        out_specs=_qkvs_out_specs(),
        out_shape=_qkvs_out_shapes(),
    )(acc, sprev, w, b)


def _pool(acc, sprev):
    return pl.pallas_call(
        _pool_body,
        grid=(GRID_N,),
        in_specs=[
            pl.BlockSpec((2, HEADS, BN, 32), lambda i: (0, 0, i, 0)),
            pl.BlockSpec((BN, 64), lambda i: (i, 0)),
        ],
        out_specs=pl.BlockSpec((1, 1, 64), lambda i: (i, 0, 0)),
        out_shape=jax.ShapeDtypeStruct((NUM_UAVS, 1, 64), jnp.float32),
    )(acc, sprev).reshape(NUM_UAVS, 64)


def _heads(pooled, uinfo, speeds, wout, bout, actor_w, critic_w):
    (wa1, ba1), (wa2, ba2), (wa3, ba3) = actor_w
    (wc1, bc1), (wc2, bc2), (wc3, bc3) = critic_w
    args = [pooled, uinfo, speeds, wout, bout.reshape(1, 64),
            wa1, ba1.reshape(1, -1), wa2, ba2.reshape(1, -1), wa3, ba3.reshape(1, -1),
            wc1, bc1.reshape(1, -1), wc2, bc2.reshape(1, -1), wc3, bc3.reshape(1, -1)]
    in_specs = [pl.BlockSpec(a.shape, lambda i: tuple(0 for _ in a.shape)) for a in args]
    return pl.pallas_call(
        _head_body,
        grid=(1,),
        in_specs=in_specs,
        out_specs=[
            pl.BlockSpec((NUM_UAVS, NUM_MISSIONS), lambda i: (0, 0)),
            pl.BlockSpec((NUM_UAVS, 1), lambda i: (0, 0)),
        ],
        out_shape=[
            jax.ShapeDtypeStruct((NUM_UAVS, NUM_MISSIONS), jnp.float32),
            jax.ShapeDtypeStruct((NUM_UAVS, 1), jnp.float32),
        ],
    )(*args)


_MESH = plsc.VectorSubcoreMesh(core_axis_name="c", subcore_axis_name="s")


@functools.partial(
    pl.kernel,
    out_type=jax.ShapeDtypeStruct((NC, HEADS, N_NODES, 32), jnp.float32),
    mesh=_MESH,
    compiler_params=pltpu.CompilerParams(
        needs_layout_passes=False, use_tc_tiling_on_sc=False),
    scratch_types=[
        pltpu.VMEM_SHARED((N_NODES, 32), jnp.float32),
        pltpu.VMEM((CHUNK, 16), jnp.float32),
        pltpu.VMEM((CHUNK, 16), jnp.float32),
        pltpu.VMEM((CHUNK, 16), jnp.float32),
        pltpu.VMEM((CHUNK, 32), jnp.float32),
        pltpu.VMEM((2, CHUNK), jnp.int32),
        pltpu.SemaphoreType.DMA,
    ],
)
def _sc_edge(q0, q1, q2, q3, k0, k1, k2, k3, v0, v1, v2, v3,
             srcp, dstp, zhbm, out, ACC, QD, KS, VS, ROW, IDX, sem):
    c = lax.axis_index("c")
    s = lax.axis_index("s")
    tid = c * NS + s
    ebase0 = tid * EDGES_PER_TILE
    nbase = s * NRANGE
    qt = [q0, q1, q2, q3]
    kt = [k0, k1, k2, k3]
    vt = [v0, v1, v2, v3]

    zero16 = jnp.zeros((16,), jnp.float32)

    def _zero_row(i, carry):
        ROW[i, 16:32] = zero16
        return carry

    lax.fori_loop(0, CHUNK, _zero_row, 0)

    is_last = s == NS - 1

    for h in range(HEADS):
        Qh, Kh, Vh = qt[h], kt[h], vt[h]

        @pl.when(jnp.logical_not(is_last))
        def _():
            pltpu.sync_copy(zhbm.at[pl.ds(0, NRANGE), :],
                            ACC.at[pl.ds(nbase, NRANGE), :])

        @pl.when(is_last)
        def _():
            pltpu.sync_copy(zhbm.at[pl.ds(0, NRANGE_LAST), :],
                            ACC.at[pl.ds(nbase, NRANGE_LAST), :])

        plsc.subcore_barrier()

        def _chunk(j, carry):
            base = ebase0 + j * CHUNK
            pltpu.sync_copy(dstp.at[pl.ds(base, CHUNK)], IDX.at[0])
            pltpu.sync_copy(srcp.at[pl.ds(base, CHUNK)], IDX.at[1])
            cq = pltpu.async_copy(Qh.at[IDX.at[0]], QD, sem)
            ck = pltpu.async_copy(Kh.at[IDX.at[1]], KS, sem)
            cv = pltpu.async_copy(Vh.at[IDX.at[1]], VS, sem)
            cq.wait()
            ck.wait()
            cv.wait()
            for g in range(8):
                rows = jnp.full((16,), g * 16, jnp.int32) + lax.iota(jnp.int32, 16)
                acc = zero16
                for jj in range(HID):
                    col = jnp.full((16,), jj, jnp.int32)
                    qc = plsc.load_gather(QD, [rows, col])
                    kc = plsc.load_gather(KS, [rows, col])
                    acc = acc + qc * kc
                ex = jnp.exp(acc)
                gi = jnp.full((16,), base + g * 16, jnp.int32) + lax.iota(jnp.int32, 16)
                ex = jnp.where(gi < N_EDGES, ex, 0.0)
                plsc.store_scatter(ROW, [rows, jnp.full((16,), 16, jnp.int32)], ex)
                for jj in range(HID):
                    col = jnp.full((16,), jj, jnp.int32)
                    vc = plsc.load_gather(VS, [rows, col])
                    plsc.store_scatter(ROW, [rows, col], vc * ex)
            pltpu.sync_copy(ROW, ACC.at[IDX.at[0]], add=True)
            return carry

        lax.fori_loop(0, CHUNKS_PER_TILE, _chunk, 0)
        plsc.subcore_barrier()

        @pl.when(jnp.logical_not(is_last))
        def _():
            pltpu.sync_copy(ACC.at[pl.ds(nbase, NRANGE), :],
                            out.at[c, h, pl.ds(nbase, NRANGE), :])

        @pl.when(is_last)
        def _():
            pltpu.sync_copy(ACC.at[pl.ds(nbase, NRANGE_LAST), :],
                            out.at[c, h, pl.ds(nbase, NRANGE_LAST), :])

        plsc.subcore_barrier()


def kernel(mission_coords, edge_index, batch, uavs_info, action_mask, speeds,
           dist_matrix, timetogo_matrix, params):
    U, M = NUM_UAVS, NUM_MISSIONS
    f32 = jnp.float32

    mask_e = action_mask.astype(f32)[..., None]
    speeds_e = jnp.broadcast_to(speeds[:, None, None], (U, M, 1))
    mc_e = jnp.broadcast_to(mission_coords[None, :, :], (U, M, 2))
    x0 = jnp.concatenate(
        [mc_e, mask_e, speeds_e, dist_matrix[..., None], timetogo_matrix[..., None]],
        axis=-1).reshape(U * M, IN_CH)
    x0 = jnp.pad(x0, ((0, 0), (0, 8 - IN_CH)))

    src = edge_index[0]
    dst = edge_index[1]
    srcp = jnp.pad(src, (0, E_PAD - N_EDGES))
    dstp = jnp.pad(dst, (0, E_PAD - N_EDGES))
    zhbm = jnp.zeros((NRANGE, 32), f32)

    Ws_fused = []
    bs_fused = []
    for p in params["convs"]:
        w = jnp.concatenate([p["Wq"] * 0.25, p["Wk"], p["Wv"], p["Ws"]], axis=1)
        b = jnp.concatenate([p["bq"] * 0.25, p["bk"], p["bv"], p["bs"]])
        Ws_fused.append(w)
        bs_fused.append(b.reshape(1, 256))
    w1 = jnp.pad(Ws_fused[0], ((0, 8 - IN_CH), (0, 0)))

    *tabs, sprev = _mm_first(x0, w1, bs_fused[0])
    acc = _sc_edge(*tabs, srcp, dstp, zhbm)

    for li in range(1, 4):
        *tabs, snew = _mm_mid(acc, sprev, Ws_fused[li], bs_fused[li])
        acc = _sc_edge(*tabs, srcp, dstp, zhbm)
        sprev = snew

    pooled = _pool(acc, sprev)
    logits, val = _heads(
        pooled, uavs_info, speeds.reshape(U, 1),
        params["Wout"], params["bout"], params["actor"], params["critic"])
    return logits, val[:, 0]


# speedup vs baseline: 35.3060x; 1.5860x over previous
import functools

import jax
import jax.numpy as jnp
import numpy as np
from jax import lax
from jax.experimental import pallas as pl
from jax.experimental.pallas import tpu as pltpu
from jax.experimental.pallas import tpu_sc as plsc

NUM_UAVS = 50
NUM_MISSIONS = 1000
N_NODES = NUM_UAVS * NUM_MISSIONS
N_EDGES = 800000
HEADS = 4
HID = 16
EMB = 64
IN_CH = 6

NC = 2
NS = 16
NTILES = NC * NS
CHUNK = 128
CHUNKS_PER_TILE = 196
EDGES_PER_TILE = CHUNK * CHUNKS_PER_TILE
E_PAD = EDGES_PER_TILE * NTILES
NRANGE = 3128
NRANGE_LAST = N_NODES - 15 * NRANGE
BN = 1000
GRID_N = N_NODES // BN


def _write_qkvs(y, outs):
    oq, ok, ov, os_ = outs
    for h in range(HEADS):
        oq[h] = y[:, 16 * h:16 * h + 16]
        ok[h] = y[:, 64 + 16 * h:64 + 16 * h + 16]
        ov[h] = y[:, 128 + 16 * h:128 + 16 * h + 16]
    os_[...] = y[:, 192:256]


def _mm_first_body(x_ref, w_ref, b_ref, *outs):
    y = jnp.dot(x_ref[...], w_ref[...], preferred_element_type=jnp.float32, precision=lax.Precision.HIGHEST)
    y = y + b_ref[...]
    _write_qkvs(y, outs)


def _merge_acc(acc_ref, sprev_ref):
    cols = []
    for h in range(HEADS):
        a0 = acc_ref[0, h]
        a1 = acc_ref[1, h]
        num = a0[:, 0:16] + a1[:, 0:16]
        den = a0[:, 16:17] + a1[:, 16:17]
        cols.append(num / (den + 1e-16))
    x = jnp.concatenate(cols, axis=1) + sprev_ref[...]
    return jnp.maximum(x, 0.0)


def _mm_mid_body(acc_ref, sprev_ref, w_ref, b_ref, *outs):
    x = _merge_acc(acc_ref, sprev_ref)
    y = jnp.dot(x, w_ref[...], preferred_element_type=jnp.float32, precision=lax.Precision.HIGHEST) + b_ref[...]
    _write_qkvs(y, outs)


def _pool_body(acc_ref, sprev_ref, out_ref):
    x = _merge_acc(acc_ref, sprev_ref)
    p = jnp.sum(x, axis=0, keepdims=True) * (1.0 / NUM_MISSIONS)
    out_ref[...] = p.reshape(1, 1, 64)


def _head_body(pooled_ref, uinfo_ref, speeds_ref,
               wout_ref, bout_ref,
               wa1_ref, ba1_ref, wa2_ref, ba2_ref, wa3_ref, ba3_ref,
               wc1_ref, bc1_ref, wc2_ref, bc2_ref, wc3_ref, bc3_ref,
               logits_ref, val_ref):
    emb = jnp.dot(pooled_ref[...], wout_ref[...],
                  preferred_element_type=jnp.float32, precision=lax.Precision.HIGHEST) + bout_ref[...]
    comb = jnp.concatenate([uinfo_ref[...], emb, speeds_ref[...]], axis=1)

    h = jnp.maximum(jnp.dot(comb, wa1_ref[...], preferred_element_type=jnp.float32, precision=lax.Precision.HIGHEST) + ba1_ref[...], 0.0)
    h = jnp.maximum(jnp.dot(h, wa2_ref[...], preferred_element_type=jnp.float32, precision=lax.Precision.HIGHEST) + ba2_ref[...], 0.0)
    logits_ref[...] = jnp.dot(h, wa3_ref[...], preferred_element_type=jnp.float32, precision=lax.Precision.HIGHEST) + ba3_ref[...]

    h = jnp.maximum(jnp.dot(comb, wc1_ref[...], preferred_element_type=jnp.float32, precision=lax.Precision.HIGHEST) + bc1_ref[...], 0.0)
    h = jnp.maximum(jnp.dot(h, wc2_ref[...], preferred_element_type=jnp.float32, precision=lax.Precision.HIGHEST) + bc2_ref[...], 0.0)
    val_ref[...] = jnp.dot(h, wc3_ref[...], preferred_element_type=jnp.float32, precision=lax.Precision.HIGHEST) + bc3_ref[...]


def _qkvs_out_shapes():
    shapes = [jax.ShapeDtypeStruct((HEADS, N_NODES, 16), jnp.float32) for _ in range(3)]
    shapes.append(jax.ShapeDtypeStruct((N_NODES, 64), jnp.float32))
    return shapes


def _qkvs_out_specs():
    specs = [pl.BlockSpec((HEADS, BN, 16), lambda i: (0, i, 0)) for _ in range(3)]
    specs.append(pl.BlockSpec((BN, 64), lambda i: (i, 0)))
    return specs


def _mm_first(x, w, b):
    return pl.pallas_call(
        _mm_first_body,
        grid=(GRID_N,),
        in_specs=[
            pl.BlockSpec((BN, 8), lambda i: (i, 0)),
            pl.BlockSpec((8, 256), lambda i: (0, 0)),
            pl.BlockSpec((1, 256), lambda i: (0, 0)),
        ],
        out_specs=_qkvs_out_specs(),
        out_shape=_qkvs_out_shapes(),
    )(x, w, b)


def _mm_mid(acc, sprev, w, b):
    return pl.pallas_call(
        _mm_mid_body,
        grid=(GRID_N,),
        in_specs=[
            pl.BlockSpec((2, HEADS, BN, 32), lambda i: (0, 0, i, 0)),
            pl.BlockSpec((BN, 64), lambda i: (i, 0)),
            pl.BlockSpec((64, 256), lambda i: (0, 0)),
            pl.BlockSpec((1, 256), lambda i: (0, 0)),
        ],
        out_specs=_qkvs_out_specs(),
        out_shape=_qkvs_out_shapes(),
    )(acc, sprev, w, b)


def _pool(acc, sprev):
    return pl.pallas_call(
        _pool_body,
        grid=(GRID_N,),
        in_specs=[
            pl.BlockSpec((2, HEADS, BN, 32), lambda i: (0, 0, i, 0)),
            pl.BlockSpec((BN, 64), lambda i: (i, 0)),
        ],
        out_specs=pl.BlockSpec((1, 1, 64), lambda i: (i, 0, 0)),
        out_shape=jax.ShapeDtypeStruct((NUM_UAVS, 1, 64), jnp.float32),
    )(acc, sprev).reshape(NUM_UAVS, 64)


def _heads(pooled, uinfo, speeds, wout, bout, actor_w, critic_w):
    (wa1, ba1), (wa2, ba2), (wa3, ba3) = actor_w
    (wc1, bc1), (wc2, bc2), (wc3, bc3) = critic_w
    args = [pooled, uinfo, speeds, wout, bout.reshape(1, 64),
            wa1, ba1.reshape(1, -1), wa2, ba2.reshape(1, -1), wa3, ba3.reshape(1, -1),
            wc1, bc1.reshape(1, -1), wc2, bc2.reshape(1, -1), wc3, bc3.reshape(1, -1)]
    in_specs = [pl.BlockSpec(a.shape, lambda i: tuple(0 for _ in a.shape)) for a in args]
    return pl.pallas_call(
        _head_body,
        grid=(1,),
        in_specs=in_specs,
        out_specs=[
            pl.BlockSpec((NUM_UAVS, NUM_MISSIONS), lambda i: (0, 0)),
            pl.BlockSpec((NUM_UAVS, 1), lambda i: (0, 0)),
        ],
        out_shape=[
            jax.ShapeDtypeStruct((NUM_UAVS, NUM_MISSIONS), jnp.float32),
            jax.ShapeDtypeStruct((NUM_UAVS, 1), jnp.float32),
        ],
    )(*args)


_MESH = plsc.VectorSubcoreMesh(core_axis_name="c", subcore_axis_name="s")

SB = 14
NSB = CHUNKS_PER_TILE // SB


@functools.partial(
    pl.kernel,
    out_type=jax.ShapeDtypeStruct((NC, HEADS, N_NODES, 32), jnp.float32),
    mesh=_MESH,
    compiler_params=pltpu.CompilerParams(
        needs_layout_passes=False, use_tc_tiling_on_sc=False),
    scratch_types=[
        pltpu.VMEM_SHARED((N_NODES, 32), jnp.float32),
        pltpu.VMEM((2 * CHUNK, 16), jnp.float32),
        pltpu.VMEM((2 * CHUNK, 16), jnp.float32),
        pltpu.VMEM((2 * CHUNK, 16), jnp.float32),
        pltpu.VMEM((2 * CHUNK, 32), jnp.float32),
        pltpu.VMEM((2, CHUNK), jnp.int32),
        pltpu.VMEM((2, 2, SB, CHUNK), jnp.int32),
        pltpu.SemaphoreType.DMA,
        pltpu.SemaphoreType.DMA,
        pltpu.SemaphoreType.DMA,
        pltpu.SemaphoreType.DMA,
        pltpu.SemaphoreType.DMA,
    ],
)
def _sc_edge(q, k, v, srcp, dstp, zhbm, out,
             ACC, QD, KS, VS, ROW, SIDX, IDXALL,
             semg0, semg1, sems0, sems1, semi):
    c = lax.axis_index("c")
    s = lax.axis_index("s")
    tid = c * NS + s
    rbase0 = tid * CHUNKS_PER_TILE
    nbase = s * NRANGE
    semg = [semg0, semg1]
    sems = [sems0, sems1]

    zero16 = jnp.zeros((16,), jnp.float32)
    izero16 = jnp.zeros((16,), jnp.int32)
    iota16 = lax.iota(jnp.int32, 16)
    i16_16 = jnp.full((16,), 16, jnp.int32)
    cols = [jnp.full((16,), jj, jnp.int32) for jj in range(HID)]
    is_last = s == NS - 1

    def _issue_idx(sb):
        pi = lax.rem(sb, 2)
        rb = rbase0 + sb * SB
        pltpu.async_copy(dstp.at[pl.ds(rb, SB), :], IDXALL.at[pi, 0], semi)
        pltpu.async_copy(srcp.at[pl.ds(rb, SB), :], IDXALL.at[pi, 1], semi)

    def _wait_idx(sb):
        pi = lax.rem(sb, 2)
        rb = rbase0 + sb * SB
        pltpu.make_async_copy(dstp.at[pl.ds(rb, SB), :], IDXALL.at[pi, 0], semi).wait()
        pltpu.make_async_copy(srcp.at[pl.ds(rb, SB), :], IDXALL.at[pi, 1], semi).wait()

    def _issue_gathers(h, j1, b1):
        sb1 = lax.div(j1, SB)
        cc1 = lax.rem(j1, SB)
        pi1 = lax.rem(sb1, 2)
        idxd = IDXALL.at[pi1, 0, cc1]
        idxs = IDXALL.at[pi1, 1, cc1]

        @pl.when(b1 == 0)
        def _():
            pltpu.async_copy(q.at[h].at[idxd], QD.at[pl.ds(0, CHUNK), :], semg0)
            pltpu.async_copy(k.at[h].at[idxs], KS.at[pl.ds(0, CHUNK), :], semg0)
            pltpu.async_copy(v.at[h].at[idxs], VS.at[pl.ds(0, CHUNK), :], semg0)

        @pl.when(b1 == 1)
        def _():
            pltpu.async_copy(q.at[h].at[idxd], QD.at[pl.ds(CHUNK, CHUNK), :], semg1)
            pltpu.async_copy(k.at[h].at[idxs], KS.at[pl.ds(CHUNK, CHUNK), :], semg1)
            pltpu.async_copy(v.at[h].at[idxs], VS.at[pl.ds(CHUNK, CHUNK), :], semg1)

    def _wait_gathers(h, b):
        @pl.when(b == 0)
        def _():
            pltpu.make_async_copy(q.at[h].at[SIDX.at[0]], QD.at[pl.ds(0, CHUNK), :], semg0).wait()
            pltpu.make_async_copy(q.at[h].at[SIDX.at[0]], KS.at[pl.ds(0, CHUNK), :], semg0).wait()
            pltpu.make_async_copy(q.at[h].at[SIDX.at[0]], VS.at[pl.ds(0, CHUNK), :], semg0).wait()

        @pl.when(b == 1)
        def _():
            pltpu.make_async_copy(q.at[h].at[SIDX.at[0]], QD.at[pl.ds(CHUNK, CHUNK), :], semg1).wait()
            pltpu.make_async_copy(q.at[h].at[SIDX.at[0]], KS.at[pl.ds(CHUNK, CHUNK), :], semg1).wait()
            pltpu.make_async_copy(q.at[h].at[SIDX.at[0]], VS.at[pl.ds(CHUNK, CHUNK), :], semg1).wait()

    def _issue_scatter(b):
        @pl.when(b == 0)
        def _():
            pltpu.async_copy(ROW.at[pl.ds(0, CHUNK), :], ACC.at[SIDX.at[0]], sems0, add=True)

        @pl.when(b == 1)
        def _():
            pltpu.async_copy(ROW.at[pl.ds(CHUNK, CHUNK), :], ACC.at[SIDX.at[1]], sems1, add=True)

    def _wait_scatter(b):
        @pl.when(b == 0)
        def _():
            pltpu.make_async_copy(ROW.at[pl.ds(0, CHUNK), :], ACC.at[SIDX.at[0]], sems0).wait()

        @pl.when(b == 1)
        def _():
            pltpu.make_async_copy(ROW.at[pl.ds(CHUNK, CHUNK), :], ACC.at[SIDX.at[1]], sems1).wait()

    def _compute(b, j):
        sb = lax.div(j, SB)
        cc = lax.rem(j, SB)
        pi = lax.rem(sb, 2)
        for gg in range(8):
            SIDX[b, gg * 16:(gg + 1) * 16] = IDXALL[pi, 0, cc, gg * 16:(gg + 1) * 16]
        boff = b * CHUNK
        base = (rbase0 + j) * CHUNK
        for g in range(8):
            rows = jnp.full((16,), g * 16, jnp.int32) + iota16 + boff
            accs = [zero16, zero16, zero16, zero16]
            for jj in range(HID):
                qc = plsc.load_gather(QD, [rows, cols[jj]])
                kc = plsc.load_gather(KS, [rows, cols[jj]])
                accs[jj % 4] = accs[jj % 4] + qc * kc
            acc = (accs[0] + accs[1]) + (accs[2] + accs[3])
            ex = jnp.exp(acc)
            gi = jnp.full((16,), g * 16, jnp.int32) + iota16 + base
            ex = jnp.where(gi < N_EDGES, ex, 0.0)
            plsc.store_scatter(ROW, [rows, i16_16], ex)
            for jj in range(HID):
                vc = plsc.load_gather(VS, [rows, cols[jj]])
                plsc.store_scatter(ROW, [rows, cols[jj]], vc * ex)

    def _zero_rows(i, carry):
        ROW[i, 0:16] = zero16
        ROW[i, 16:32] = zero16
        return carry

    def _head(h, carry):
        lax.fori_loop(0, 2 * CHUNK, _zero_rows, 0)
        for gg in range(8):
            SIDX[0, gg * 16:(gg + 1) * 16] = izero16
            SIDX[1, gg * 16:(gg + 1) * 16] = izero16

        @pl.when(jnp.logical_not(is_last))
        def _():
            pltpu.sync_copy(zhbm.at[pl.ds(0, NRANGE), :],
                            ACC.at[pl.ds(nbase, NRANGE), :])

        @pl.when(is_last)
        def _():
            pltpu.sync_copy(zhbm.at[pl.ds(0, NRANGE_LAST), :],
                            ACC.at[pl.ds(nbase, NRANGE_LAST), :])

        plsc.subcore_barrier()

        _issue_idx(0)
        _wait_idx(0)
        _issue_scatter(0)
        _issue_scatter(1)
        _issue_gathers(h, 0, 0)

        def _body(j, carry2):
            b = lax.rem(j, 2)
            bn = 1 - b
            jn = jnp.minimum(j + 1, CHUNKS_PER_TILE - 1)
            sb = lax.div(j, SB)
            cc = lax.rem(j, SB)

            @pl.when(jnp.logical_and(cc == 0, sb < NSB - 1))
            def _():
                _issue_idx(sb + 1)

            @pl.when(jnp.logical_and(cc == SB - 1, sb < NSB - 1))
            def _():
                _wait_idx(sb + 1)

            _issue_gathers(h, jn, bn)
            _wait_scatter(b)
            _wait_gathers(h, b)
            _compute(b, j)
            _issue_scatter(b)
            return carry2

        lax.fori_loop(0, CHUNKS_PER_TILE, _body, 0)

        _wait_gathers(h, 0)
        _wait_scatter(0)
        _wait_scatter(1)
        plsc.subcore_barrier()

        @pl.when(jnp.logical_not(is_last))
        def _():
            pltpu.sync_copy(ACC.at[pl.ds(nbase, NRANGE), :],
                            out.at[c, h, pl.ds(nbase, NRANGE), :])

        @pl.when(is_last)
        def _():
            pltpu.sync_copy(ACC.at[pl.ds(nbase, NRANGE_LAST), :],
                            out.at[c, h, pl.ds(nbase, NRANGE_LAST), :])

        plsc.subcore_barrier()
        return carry

    lax.fori_loop(0, HEADS, _head, 0)


def kernel(mission_coords, edge_index, batch, uavs_info, action_mask, speeds,
           dist_matrix, timetogo_matrix, params):
    U, M = NUM_UAVS, NUM_MISSIONS
    f32 = jnp.float32

    mask_e = action_mask.astype(f32)[..., None]
    speeds_e = jnp.broadcast_to(speeds[:, None, None], (U, M, 1))
    mc_e = jnp.broadcast_to(mission_coords[None, :, :], (U, M, 2))
    x0 = jnp.concatenate(
        [mc_e, mask_e, speeds_e, dist_matrix[..., None], timetogo_matrix[..., None]],
        axis=-1).reshape(U * M, IN_CH)
    x0 = jnp.pad(x0, ((0, 0), (0, 8 - IN_CH)))

    src = edge_index[0]
    dst = edge_index[1]
    srcp = jnp.pad(src, (0, E_PAD - N_EDGES)).reshape(E_PAD // CHUNK, CHUNK)
    dstp = jnp.pad(dst, (0, E_PAD - N_EDGES)).reshape(E_PAD // CHUNK, CHUNK)
    zhbm = jnp.zeros((NRANGE, 32), f32)

    Ws_fused = []
    bs_fused = []
    for p in params["convs"]:
        w = jnp.concatenate([p["Wq"] * 0.25, p["Wk"], p["Wv"], p["Ws"]], axis=1)
        b = jnp.concatenate([p["bq"] * 0.25, p["bk"], p["bv"], p["bs"]])
        Ws_fused.append(w)
        bs_fused.append(b.reshape(1, 256))
    w1 = jnp.pad(Ws_fused[0], ((0, 8 - IN_CH), (0, 0)))

    *tabs, sprev = _mm_first(x0, w1, bs_fused[0])
    acc = _sc_edge(*tabs, srcp, dstp, zhbm)

    for li in range(1, 4):
        *tabs, snew = _mm_mid(acc, sprev, Ws_fused[li], bs_fused[li])
        acc = _sc_edge(*tabs, srcp, dstp, zhbm)
        sprev = snew

    pooled = _pool(acc, sprev)
    logits, val = _heads(
        pooled, uavs_info, speeds.reshape(U, 1),
        params["Wout"], params["bout"], params["actor"], params["critic"])
    return logits, val[:, 0]
